# baseline (speedup 1.0000x reference)
import jax
import jax.numpy as jnp
from jax import lax
from jax.experimental import pallas as pl
from jax.experimental.pallas import tpu as pltpu
from jax.experimental.pallas import tpu_sc as plsc

B = 16384
DIM = 64
PDIM = 2 * DIM
NC = 2
NS = 16
NW = NC * NS
BPW = B // NW
CHUNK = 64
NCHUNK = BPW // CHUNK
NTASK = 2 * NCHUNK


def _score_chunk(s_v, r_v, t_v, o_v, sp_v, rp_v, tp_v, op_v, out_v, out_base):
    lane = lax.iota(jnp.int32, 16)

    def group(g, carry):
        gb = out_base + g * 16
        ir = g * 16 + lane
        sp = sp_v[pl.ds(gb, 16)] * DIM
        rp = rp_v[pl.ds(gb, 16)] * DIM
        tp = tp_v[pl.ds(gb, 16)] * DIM
        op = op_v[pl.ds(gb, 16)] * DIM

        def dim_body(d, acc):
            sv = plsc.load_gather(s_v, [ir, sp + d])
            rv = plsc.load_gather(r_v, [ir, rp + d])
            tv = plsc.load_gather(t_v, [ir, tp + d])
            ov = plsc.load_gather(o_v, [ir, op + d])
            return acc + jnp.abs(sv + rv + tv - ov)

        acc = lax.fori_loop(0, DIM, dim_body, jnp.zeros((16,), jnp.float32))
        out_v[pl.ds(gb, 16)] = acc
        return carry

    lax.fori_loop(0, CHUNK // 16, group, 0)


def _body(ps_h, pr_h, po_h, pt_h, ns_h, nr_h, no_h,
          psp_h, prp_h, pop_h, ptp_h, nsp_h, nrp_h, nop_h,
          e_w, r_w, t_w, pos_out, neg_out,
          ps_v, pr_v, po_v, pt_v, ns_v, nr_v, no_v,
          psp_v, prp_v, pop_v, ptp_v, nsp_v, nrp_v, nop_v,
          s_v0, r_v0, t_v0, o_v0, s_v1, r_v1, t_v1, o_v1,
          pos_ov, neg_ov, sem0, sem1):
    wid = lax.axis_index("s") * NC + lax.axis_index("c")
    base = wid * BPW

    for hbm, vmem in ((ps_h, ps_v), (pr_h, pr_v), (po_h, po_v),
                      (pt_h, pt_v), (ns_h, ns_v), (nr_h, nr_v),
                      (no_h, no_v), (psp_h, psp_v), (prp_h, prp_v),
                      (pop_h, pop_v), (ptp_h, ptp_v), (nsp_h, nsp_v),
                      (nrp_h, nrp_v), (nop_h, nop_v)):
        pltpu.sync_copy(hbm.at[pl.ds(base, BPW)], vmem)

    bufs = ((s_v0, r_v0, t_v0, o_v0), (s_v1, r_v1, t_v1, o_v1))
    sems = (sem0, sem1)
    tasks = [(c, (ps_v, pr_v, pt_v, po_v), (psp_v, prp_v, ptp_v, pop_v),
              pos_ov) for c in range(NCHUNK)]
    tasks += [(c, (ns_v, nr_v, pt_v, no_v), (nsp_v, nrp_v, ptp_v, nop_v),
               neg_ov) for c in range(NCHUNK)]

    def fire(k):
        c, (si, ri, ti, oi), _, _ = tasks[k]
        sb, rb, tb, ob = bufs[k % 2]
        sem = sems[k % 2]
        cb = c * CHUNK
        return (pltpu.async_copy(e_w.at[si.at[pl.ds(cb, CHUNK)]], sb, sem),
                pltpu.async_copy(r_w.at[ri.at[pl.ds(cb, CHUNK)]], rb, sem),
                pltpu.async_copy(t_w.at[ti.at[pl.ds(cb, CHUNK)]], tb, sem),
                pltpu.async_copy(e_w.at[oi.at[pl.ds(cb, CHUNK)]], ob, sem))

    pending = fire(0)
    for k in range(NTASK):
        for cp in pending:
            cp.wait()
        if k + 1 < NTASK:
            nxt = fire(k + 1)
        c, _, (spv, rpv, tpv, opv), out_v = tasks[k]
        sb, rb, tb, ob = bufs[k % 2]
        _score_chunk(sb, rb, tb, ob, spv, rpv, tpv, opv, out_v, c * CHUNK)
        if k + 1 < NTASK:
            pending = nxt

    pltpu.sync_copy(pos_ov, pos_out.at[pl.ds(base, BPW)])
    pltpu.sync_copy(neg_ov, neg_out.at[pl.ds(base, BPW)])


def kernel(pos_s, pos_r, pos_o, pos_t, neg_s, neg_r, neg_o, neg_t,
           e_weight, r_weight, t_weight):
    mesh = plsc.VectorSubcoreMesh(
        core_axis_name="c", subcore_axis_name="s",
        num_cores=NC, num_subcores=NS)
    f32 = jnp.float32
    i32 = jnp.int32
    run = pl.kernel(
        _body,
        out_type=(jax.ShapeDtypeStruct((B,), f32),
                  jax.ShapeDtypeStruct((B,), f32)),
        mesh=mesh,
        scratch_types=(
            [pltpu.VMEM((BPW,), i32)] * 14
            + [pltpu.VMEM((CHUNK, PDIM), f32)] * 8
            + [pltpu.VMEM((BPW,), f32)] * 2
            + [pltpu.SemaphoreType.DMA] * 2
        ),
        compiler_params=pltpu.CompilerParams(needs_layout_passes=False),
    )

    def prep(ix):
        ix = ix.astype(i32)
        return ix // 2, ix & 1

    psi, psp = prep(pos_s)
    pri, prp = prep(pos_r)
    poi, pop = prep(pos_o)
    pti, ptp = prep(pos_t)
    nsi, nsp = prep(neg_s)
    nri, nrp = prep(neg_r)
    noi, nop = prep(neg_o)
    e_p = jnp.concatenate([e_weight[0::2], e_weight[1::2]], axis=1)
    r_p = r_weight.reshape(r_weight.shape[0] // 2, PDIM)
    t_p = t_weight.reshape(t_weight.shape[0] // 2, PDIM)
    return run(psi, pri, poi, pti, nsi, nri, noi,
               psp, prp, pop, ptp, nsp, nrp, nop,
               e_p, r_p, t_p)


# speedup vs baseline: 13.9671x; 13.9671x over previous
import jax
import jax.numpy as jnp
from jax import lax
from jax.experimental import pallas as pl
from jax.experimental.pallas import tpu as pltpu
from jax.experimental.pallas import tpu_sc as plsc

B = 16384
DIM = 64
NC = 2
NS = 16
NW = NC * NS
BPW = B // NW
CHUNK = 128
NCHUNK = BPW // CHUNK
NTASK = 2 * NCHUNK


def _score_chunk(s_v, r_v, t_v, o_v, out_v, out_base):
    lane = lax.iota(jnp.int32, 16)
    last = lane == 15

    def elem(i, carry):
        total = jnp.zeros((16,), jnp.float32)
        for q in range(DIM // 16):
            sl = pl.ds(q * 16, 16)
            total = total + jnp.abs(
                s_v[i, sl] + r_v[i, sl] + t_v[i, sl] - o_v[i, sl])
        csum = plsc.cumsum(total)
        idx = jnp.full((16,), out_base + i, jnp.int32)
        plsc.store_scatter(out_v, [idx], csum, mask=last)
        return carry

    lax.fori_loop(0, CHUNK, elem, 0)


def _body(pos_s, pos_r, pos_o, pos_t, neg_s, neg_r, neg_o, neg_t,
          e_w, r_w, t_w, pos_out, neg_out,
          ps_v, pr_v, po_v, pt_v, ns_v, nr_v, no_v,
          s_v0, r_v0, t_v0, o_v0, s_v1, r_v1, t_v1, o_v1,
          pos_ov, neg_ov, sem0, sem1):
    wid = lax.axis_index("s") * NC + lax.axis_index("c")
    base = wid * BPW

    for hbm, vmem in ((pos_s, ps_v), (pos_r, pr_v), (pos_o, po_v),
                      (pos_t, pt_v), (neg_s, ns_v), (neg_r, nr_v),
                      (neg_o, no_v)):
        pltpu.sync_copy(hbm.at[pl.ds(base, BPW)], vmem)

    bufs = ((s_v0, r_v0, t_v0, o_v0), (s_v1, r_v1, t_v1, o_v1))
    sems = (sem0, sem1)
    tasks = [(c, (ps_v, pr_v, pt_v, po_v), pos_ov) for c in range(NCHUNK)]
    tasks += [(c, (ns_v, nr_v, pt_v, no_v), neg_ov) for c in range(NCHUNK)]

    def fire(k):
        c, (si, ri, ti, oi), _ = tasks[k]
        sb, rb, tb, ob = bufs[k % 2]
        sem = sems[k % 2]
        cb = c * CHUNK
        return (pltpu.async_copy(e_w.at[si.at[pl.ds(cb, CHUNK)]], sb, sem),
                pltpu.async_copy(r_w.at[ri.at[pl.ds(cb, CHUNK)]], rb, sem),
                pltpu.async_copy(t_w.at[ti.at[pl.ds(cb, CHUNK)]], tb, sem),
                pltpu.async_copy(e_w.at[oi.at[pl.ds(cb, CHUNK)]], ob, sem))

    pending = fire(0)
    for k in range(NTASK):
        for cp in pending:
            cp.wait()
        if k + 1 < NTASK:
            nxt = fire(k + 1)
        c, _, out_v = tasks[k]
        sb, rb, tb, ob = bufs[k % 2]
        _score_chunk(sb, rb, tb, ob, out_v, c * CHUNK)
        if k + 1 < NTASK:
            pending = nxt

    pltpu.sync_copy(pos_ov, pos_out.at[pl.ds(base, BPW)])
    pltpu.sync_copy(neg_ov, neg_out.at[pl.ds(base, BPW)])


def kernel(pos_s, pos_r, pos_o, pos_t, neg_s, neg_r, neg_o, neg_t,
           e_weight, r_weight, t_weight):
    mesh = plsc.VectorSubcoreMesh(
        core_axis_name="c", subcore_axis_name="s",
        num_cores=NC, num_subcores=NS)
    f32 = jnp.float32
    run = pl.kernel(
        _body,
        out_type=(jax.ShapeDtypeStruct((B,), f32),
                  jax.ShapeDtypeStruct((B,), f32)),
        mesh=mesh,
        scratch_types=(
            [pltpu.VMEM((BPW,), jnp.int32)] * 7
            + [pltpu.VMEM((CHUNK, DIM), f32)] * 8
            + [pltpu.VMEM((BPW,), f32)] * 2
            + [pltpu.SemaphoreType.DMA] * 2
        ),
        compiler_params=pltpu.CompilerParams(
            needs_layout_passes=False, use_tc_tiling_on_sc=False),
    )
    return run(pos_s.astype(jnp.int32), pos_r.astype(jnp.int32),
               pos_o.astype(jnp.int32), pos_t.astype(jnp.int32),
               neg_s.astype(jnp.int32), neg_r.astype(jnp.int32),
               neg_o.astype(jnp.int32), neg_t.astype(jnp.int32),
               e_weight, r_weight, t_weight)
